# Initial kernel scaffold; baseline (speedup 1.0000x reference)
import jax
import jax.numpy as jnp
from jax.experimental import pallas as pl


def kernel(x, edge_index, edge_attr, params):
    raise NotImplementedError("write your pallas kernel here")


# speedup vs baseline: 2.2224x; 2.2224x over previous
import functools

import jax
import jax.numpy as jnp
from jax import lax
from jax.experimental import pallas as pl
from jax.experimental.pallas import tpu as pltpu
from jax.experimental.pallas import tpu_sc as plsc

_N = 10000
_E = 320000
_H = 128
_DE = 16

_NC = 2
_NS = 16
_NW = _NC * _NS
_EW = _E // _NW
_CH = 80
_NCHUNK = _EW // _CH
_NP = 10240
_TS = _NP // _NS

_BE = 4000
_GE = _E // _BE

def _sc_mesh():
    return plsc.VectorSubcoreMesh(core_axis_name="c", subcore_axis_name="s",
                                  num_cores=_NC, num_subcores=_NS)


@functools.cache
def _sc_gather_kernel():
    @functools.partial(
        pl.kernel,
        out_type=[
            jax.ShapeDtypeStruct((_E, 2 * _H), jnp.float32),
            jax.ShapeDtypeStruct((_E, _H), jnp.float32),
        ],
        mesh=_sc_mesh(),
        scratch_types=[
            pltpu.VMEM((_CH,), jnp.int32),
            pltpu.VMEM((_CH,), jnp.int32),
            pltpu.VMEM((_CH, 2 * _H), jnp.float32),
            pltpu.VMEM((_CH, _H), jnp.float32),
            pltpu.SemaphoreType.DMA,
            pltpu.SemaphoreType.DMA,
        ],
    )
    def k(row_h, col_h, prow_h, pcol_h, grow_h, gcol_h,
          ir, ic, br, bc, sr, sc):
        wid = lax.axis_index("s") * _NC + lax.axis_index("c")
        base = wid * _EW

        def body(j, carry):
            off = base + j * _CH
            pltpu.sync_copy(row_h.at[pl.ds(off, _CH)], ir)
            pltpu.sync_copy(col_h.at[pl.ds(off, _CH)], ic)
            d1 = pltpu.async_copy(prow_h.at[ir], br, sr)
            d2 = pltpu.async_copy(pcol_h.at[ic], bc, sc)
            d1.wait()
            d2.wait()
            pltpu.sync_copy(br, grow_h.at[pl.ds(off, _CH)])
            pltpu.sync_copy(bc, gcol_h.at[pl.ds(off, _CH)])
            return carry

        lax.fori_loop(0, _NCHUNK, body, 0)

    return k


def _sc_gather(row, col, prow, pcol):
    return _sc_gather_kernel()(row, col, prow, pcol)


@functools.cache
def _sc_scatter_kernel():
    @functools.partial(
        pl.kernel,
        out_type=jax.ShapeDtypeStruct((_NC * _NP, _H), jnp.float32),
        mesh=_sc_mesh(),
        scratch_types=[
            pltpu.VMEM((_CH,), jnp.int32),
            pltpu.VMEM((_CH, _H), jnp.float32),
            pltpu.VMEM_SHARED((_NP, _H), jnp.float32),
        ],
    )
    def k(m_h, col_h, z_h, out_h, ic, bm, acc):
        cid = lax.axis_index("c")
        sid = lax.axis_index("s")
        wid = sid * _NC + cid
        pltpu.sync_copy(z_h.at[pl.ds(sid * _TS, _TS)],
                        acc.at[pl.ds(sid * _TS, _TS)])
        plsc.subcore_barrier()

        def body(j, carry):
            off = wid * _EW + j * _CH
            pltpu.sync_copy(col_h.at[pl.ds(off, _CH)], ic)
            pltpu.sync_copy(m_h.at[pl.ds(off, _CH)], bm)
            pltpu.sync_copy(bm, acc.at[ic], add=True)
            return carry

        lax.fori_loop(0, _NCHUNK, body, 0)
        plsc.subcore_barrier()
        pltpu.sync_copy(acc.at[pl.ds(sid * _TS, _TS)],
                        out_h.at[pl.ds(cid * _NP + sid * _TS, _TS)])

    return k


def _sc_scatter(m, col, zeros_np):
    return _sc_scatter_kernel()(m, col, zeros_np)


@functools.cache
def _sc_count_kernel():
    @functools.partial(
        pl.kernel,
        out_type=jax.ShapeDtypeStruct((_NC * _NP, _H), jnp.float32),
        mesh=_sc_mesh(),
        scratch_types=[
            pltpu.VMEM((_CH,), jnp.int32),
            pltpu.VMEM((_CH, _H), jnp.float32),
            pltpu.VMEM_SHARED((_NP, _H), jnp.float32),
        ],
    )
    def k(col_h, z_h, ones_h, out_h, ic, bo, acc):
        cid = lax.axis_index("c")
        sid = lax.axis_index("s")
        wid = sid * _NC + cid
        pltpu.sync_copy(z_h.at[pl.ds(sid * _TS, _TS)],
                        acc.at[pl.ds(sid * _TS, _TS)])
        pltpu.sync_copy(ones_h, bo)
        plsc.subcore_barrier()

        def body(j, carry):
            off = wid * _EW + j * _CH
            pltpu.sync_copy(col_h.at[pl.ds(off, _CH)], ic)
            pltpu.sync_copy(bo, acc.at[ic], add=True)
            return carry

        lax.fori_loop(0, _NCHUNK, body, 0)
        plsc.subcore_barrier()
        pltpu.sync_copy(acc.at[pl.ds(sid * _TS, _TS)],
                        out_h.at[pl.ds(cid * _NP + sid * _TS, _TS)])

    return k


def _sc_count(col, zeros_np, ones_ch):
    return _sc_count_kernel()(col, zeros_np, ones_ch)


def _edge_body(first, e_ref, grow_ref, gcol_ref, w0_ref, b0_ref,
               we_ref, be_ref, ae_ref, b1_ref, eo_ref, mo_ref, so_ref):
    @pl.when(pl.program_id(0) == 0)
    def _():
        so_ref[...] = jnp.zeros_like(so_ref)

    if first:
        e = jnp.maximum(
            jnp.dot(e_ref[...], w0_ref[...], preferred_element_type=jnp.float32)
            + b0_ref[...], 0.0)
    else:
        st = w0_ref[...]
        mu = st[0:1] / _E
        var = st[1:2] / _E - mu * mu
        scale = st[2:3] * lax.rsqrt(var + 1e-5)
        shift = b0_ref[...] - mu * scale
        e = e_ref[...] * scale + shift

    g = grow_ref[...]
    e2 = jnp.maximum(
        g[:, :_H] + gcol_ref[...]
        + jnp.dot(e, we_ref[...], preferred_element_type=jnp.float32)
        + be_ref[...], 0.0)
    m = jnp.maximum(
        g[:, _H:]
        + jnp.dot(e2, ae_ref[...], preferred_element_type=jnp.float32)
        + b1_ref[...], 0.0)
    eo_ref[...] = e2
    mo_ref[...] = m
    so_ref[...] += jnp.concatenate(
        [jnp.sum(e2, 0, keepdims=True), jnp.sum(e2 * e2, 0, keepdims=True)], 0)


def _edge_call(first, e_in, grow, gcol, w0, b0, we, be, ae, b1):
    d0 = e_in.shape[1]
    full = lambda s: pl.BlockSpec(s, lambda i: (0, 0))
    return pl.pallas_call(
        functools.partial(_edge_body, first),
        grid=(_GE,),
        in_specs=[
            pl.BlockSpec((_BE, d0), lambda i: (i, 0)),
            pl.BlockSpec((_BE, 2 * _H), lambda i: (i, 0)),
            pl.BlockSpec((_BE, _H), lambda i: (i, 0)),
            full(w0.shape),
            full(b0.shape),
            full((_H, _H)),
            full((1, _H)),
            full((_H, _H)),
            full((1, _H)),
        ],
        out_specs=[
            pl.BlockSpec((_BE, _H), lambda i: (i, 0)),
            pl.BlockSpec((_BE, _H), lambda i: (i, 0)),
            pl.BlockSpec((2, _H), lambda i: (0, 0)),
        ],
        out_shape=[
            jax.ShapeDtypeStruct((_E, _H), jnp.float32),
            jax.ShapeDtypeStruct((_E, _H), jnp.float32),
            jax.ShapeDtypeStruct((2, _H), jnp.float32),
        ],
    )(e_in, grow, gcol, w0, b0, we, be, ae, b1)


def _node0_body(x_ref, wn_ref, bn_ref, wrow_ref, wcol_ref,
                xo_ref, prow_ref, pcol_ref):
    x1 = jnp.maximum(
        jnp.dot(x_ref[...], wn_ref[...], preferred_element_type=jnp.float32)
        + bn_ref[...], 0.0)
    xo_ref[...] = x1
    prow_ref[...] = jnp.dot(x1, wrow_ref[...], preferred_element_type=jnp.float32)
    pcol_ref[...] = jnp.dot(x1, wcol_ref[...], preferred_element_type=jnp.float32)


def _node0_call(x, wn, bn, wrow, wcol):
    return pl.pallas_call(
        _node0_body,
        out_shape=[
            jax.ShapeDtypeStruct((_N, _H), jnp.float32),
            jax.ShapeDtypeStruct((_N, 2 * _H), jnp.float32),
            jax.ShapeDtypeStruct((_N, _H), jnp.float32),
        ],
    )(x, wn, bn, wrow, wcol)


def _agg_x(x_ref, sp_ref, cp_ref, bx_ref, bagg_ref, b2_ref):
    cnt = cp_ref[0:_N, 0:1] + cp_ref[_NP:_NP + _N, 0:1]
    s = sp_ref[0:_N, :] + sp_ref[_NP:_NP + _N, :]
    agg = s / jnp.maximum(cnt, 1.0)
    return jnp.maximum(
        jnp.dot(x_ref[...], bx_ref[...], preferred_element_type=jnp.float32)
        + jnp.dot(agg, bagg_ref[...], preferred_element_type=jnp.float32)
        + b2_ref[...], 0.0)


def _node_mid_body(x_ref, sp_ref, cp_ref, bx_ref, bagg_ref, b2_ref,
                   g_ref, b_ref, wrow_ref, wcol_ref,
                   xo_ref, prow_ref, pcol_ref):
    xn = _agg_x(x_ref, sp_ref, cp_ref, bx_ref, bagg_ref, b2_ref)
    mu = jnp.mean(xn, 0, keepdims=True)
    var = jnp.mean(xn * xn, 0, keepdims=True) - mu * mu
    xb = (xn - mu) * (g_ref[...] * lax.rsqrt(var + 1e-5)) + b_ref[...]
    xo_ref[...] = xb
    prow_ref[...] = jnp.dot(xb, wrow_ref[...], preferred_element_type=jnp.float32)
    pcol_ref[...] = jnp.dot(xb, wcol_ref[...], preferred_element_type=jnp.float32)


def _node_mid_call(x, sp, cp, bx, bagg, b2, g, b, wrow, wcol):
    return pl.pallas_call(
        _node_mid_body,
        out_shape=[
            jax.ShapeDtypeStruct((_N, _H), jnp.float32),
            jax.ShapeDtypeStruct((_N, 2 * _H), jnp.float32),
            jax.ShapeDtypeStruct((_N, _H), jnp.float32),
        ],
    )(x, sp, cp, bx, bagg, b2, g, b, wrow, wcol)


def _node_fin_body(x_ref, sp_ref, cp_ref, bx_ref, bagg_ref, b2_ref,
                   st_ref, rw_ref, rb_ref, out_ref):
    xn = _agg_x(x_ref, sp_ref, cp_ref, bx_ref, bagg_ref, b2_ref)
    node_r = jnp.mean(xn, 0, keepdims=True)
    edge_r = st_ref[0:1] / _E
    out_ref[...] = (
        jnp.dot(node_r, rw_ref[0:_H], preferred_element_type=jnp.float32)
        + jnp.dot(edge_r, rw_ref[_H:], preferred_element_type=jnp.float32)
        + rb_ref[...])


def _node_fin_call(x, sp, cp, bx, bagg, b2, st, rw, rb):
    return pl.pallas_call(
        _node_fin_body,
        out_shape=jax.ShapeDtypeStruct((1, 1), jnp.float32),
    )(x, sp, cp, bx, bagg, b2, st, rw, rb)


def kernel(x, edge_index, edge_attr, params):
    row = edge_index[0]
    col = edge_index[1]
    layers = params['layers']

    zeros_np = jnp.zeros((_NP, _H), jnp.float32)
    ones_ch = jnp.ones((_CH, _H), jnp.float32)
    cp = _sc_count(col, zeros_np, ones_ch)

    def r1(v):
        return v.reshape(1, -1)

    def tables(layer):
        wrow = jnp.concatenate(
            [layer['edge_W'][:_H], layer['n1_W'][:_H]], axis=1)
        return wrow, layer['edge_W'][_H:2 * _H]

    wrow, wcol = tables(layers[0])
    xc, prow, pcol = _node0_call(
        x, params['enc_node_W'], r1(params['enc_node_b']), wrow, wcol)

    e = edge_attr
    stats = None
    out = None
    for i, li in enumerate(layers):
        grow, gcol = _sc_gather(row, col, prow, pcol)
        we = li['edge_W'][2 * _H:]
        ae = li['n1_W'][_H:]
        if i == 0:
            w0 = params['enc_edge_W']
            b0 = r1(params['enc_edge_b'])
        else:
            w0 = jnp.concatenate([stats, r1(params['edge_norm_g'])], axis=0)
            b0 = r1(params['edge_norm_b'])
        e, m, stats = _edge_call(i == 0, e, grow, gcol, w0, b0,
                                 we, r1(li['edge_b']), ae, r1(li['n1_b']))
        sp = _sc_scatter(m, col, zeros_np)
        bx = li['n2_W'][:_H]
        bagg = li['n2_W'][_H:]
        if i < len(layers) - 1:
            wrow, wcol = tables(layers[i + 1])
            xc, prow, pcol = _node_mid_call(
                xc, sp, cp, bx, bagg, r1(li['n2_b']),
                r1(params['node_norm_g']), r1(params['node_norm_b']),
                wrow, wcol)
        else:
            out = _node_fin_call(
                xc, sp, cp, bx, bagg, r1(li['n2_b']),
                stats, params['reg_W'], r1(params['reg_b']))
    return out


# speedup vs baseline: 3.0635x; 1.3784x over previous
import functools

import jax
import jax.numpy as jnp
from jax import lax
from jax.experimental import pallas as pl
from jax.experimental.pallas import tpu as pltpu
from jax.experimental.pallas import tpu_sc as plsc

_N = 10000
_E = 320000
_H = 128
_DE = 16

_NC = 2
_NS = 16
_NW = _NC * _NS
_EW = _E // _NW
_CH = 80
_NCHUNK = _EW // _CH
_NP = 10240
_TS = _NP // _NS

_BE = 4000
_GE = _E // _BE

def _sc_mesh():
    return plsc.VectorSubcoreMesh(core_axis_name="c", subcore_axis_name="s",
                                  num_cores=_NC, num_subcores=_NS)


@functools.cache
def _sc_gather_kernel():
    @functools.partial(
        pl.kernel,
        out_type=[
            jax.ShapeDtypeStruct((_E, 2 * _H), jnp.float32),
            jax.ShapeDtypeStruct((_E, _H), jnp.float32),
        ],
        mesh=_sc_mesh(),
        scratch_types=[
            pltpu.VMEM((_EW,), jnp.int32),
            pltpu.VMEM((_EW,), jnp.int32),
            pltpu.VMEM((_CH, 2 * _H), jnp.float32),
            pltpu.VMEM((_CH, 2 * _H), jnp.float32),
            pltpu.VMEM((_CH, _H), jnp.float32),
            pltpu.VMEM((_CH, _H), jnp.float32),
            pltpu.SemaphoreType.DMA,
            pltpu.SemaphoreType.DMA,
            pltpu.SemaphoreType.DMA,
            pltpu.SemaphoreType.DMA,
        ],
    )
    def k(row_h, col_h, prow_h, pcol_h, grow_h, gcol_h,
          ir, ic, br0, br1, bc0, bc1, sr0, sr1, sc0, sc1):
        wid = lax.axis_index("s") * _NC + lax.axis_index("c")
        base = wid * _EW
        pltpu.sync_copy(row_h.at[pl.ds(base, _EW)], ir)
        pltpu.sync_copy(col_h.at[pl.ds(base, _EW)], ic)
        brs, bcs = (br0, br1), (bc0, bc1)
        srs, scs = (sr0, sr1), (sc0, sc1)

        def fire(j, p):
            pltpu.async_copy(prow_h.at[ir.at[pl.ds(j * _CH, _CH)]], brs[p], srs[p])
            pltpu.async_copy(pcol_h.at[ic.at[pl.ds(j * _CH, _CH)]], bcs[p], scs[p])

        def drain(p):
            pltpu.make_async_copy(
                prow_h.at[ir.at[pl.ds(0, _CH)]], brs[p], srs[p]).wait()
            pltpu.make_async_copy(
                pcol_h.at[ic.at[pl.ds(0, _CH)]], bcs[p], scs[p]).wait()

        def write(j, p):
            off = base + j * _CH
            pltpu.sync_copy(brs[p], grow_h.at[pl.ds(off, _CH)])
            pltpu.sync_copy(bcs[p], gcol_h.at[pl.ds(off, _CH)])

        fire(0, 0)

        def body(g, carry):
            j = 2 * g
            fire(j + 1, 1)
            drain(0)
            write(j, 0)
            fire(j + 2, 0)
            drain(1)
            write(j + 1, 1)
            return carry

        lax.fori_loop(0, (_NCHUNK - 1) // 2, body, 0)
        drain(0)
        write(_NCHUNK - 1, 0)

    return k


def _sc_gather(row, col, prow, pcol):
    return _sc_gather_kernel()(row, col, prow, pcol)


@functools.cache
def _sc_scatter_kernel():
    @functools.partial(
        pl.kernel,
        out_type=jax.ShapeDtypeStruct((_NC * _NP, _H), jnp.float32),
        mesh=_sc_mesh(),
        scratch_types=[
            pltpu.VMEM((_NCHUNK, _CH), jnp.int32),
            pltpu.VMEM((_CH, _H), jnp.float32),
            pltpu.VMEM((_CH, _H), jnp.float32),
            pltpu.VMEM_SHARED((_NP, _H), jnp.float32),
            pltpu.SemaphoreType.DMA,
            pltpu.SemaphoreType.DMA,
        ],
    )
    def k(m_h, col2_h, z_h, out_h, ic, bm0, bm1, acc, sm0, sm1):
        cid = lax.axis_index("c")
        sid = lax.axis_index("s")
        wid = sid * _NC + cid
        pltpu.sync_copy(z_h.at[pl.ds(sid * _TS, _TS)],
                        acc.at[pl.ds(sid * _TS, _TS)])
        pltpu.sync_copy(col2_h.at[wid], ic)
        bms, sms = (bm0, bm1), (sm0, sm1)

        def fire(j, p):
            pltpu.async_copy(
                m_h.at[pl.ds(wid * _EW + j * _CH, _CH)], bms[p], sms[p])

        def drain(p):
            pltpu.make_async_copy(m_h.at[pl.ds(0, _CH)], bms[p], sms[p]).wait()

        def scat(j, p):
            pltpu.sync_copy(bms[p], acc.at[ic.at[j]], add=True)

        plsc.subcore_barrier()
        fire(0, 0)

        def body(g, carry):
            j = 2 * g
            fire(j + 1, 1)
            drain(0)
            scat(j, 0)
            fire(j + 2, 0)
            drain(1)
            scat(j + 1, 1)
            return carry

        lax.fori_loop(0, (_NCHUNK - 1) // 2, body, 0)
        drain(0)
        scat(_NCHUNK - 1, 0)
        plsc.subcore_barrier()
        pltpu.sync_copy(acc.at[pl.ds(sid * _TS, _TS)],
                        out_h.at[pl.ds(cid * _NP + sid * _TS, _TS)])

    return k


def _sc_scatter(m, col2, zeros_np):
    return _sc_scatter_kernel()(m, col2, zeros_np)


@functools.cache
def _sc_count_kernel():
    @functools.partial(
        pl.kernel,
        out_type=jax.ShapeDtypeStruct((_NC * _NP, _H), jnp.float32),
        mesh=_sc_mesh(),
        scratch_types=[
            pltpu.VMEM((_NCHUNK, _CH), jnp.int32),
            pltpu.VMEM((_CH, _H), jnp.float32),
            pltpu.VMEM_SHARED((_NP, _H), jnp.float32),
        ],
    )
    def k(col2_h, z_h, ones_h, out_h, ic, bo, acc):
        cid = lax.axis_index("c")
        sid = lax.axis_index("s")
        wid = sid * _NC + cid
        pltpu.sync_copy(z_h.at[pl.ds(sid * _TS, _TS)],
                        acc.at[pl.ds(sid * _TS, _TS)])
        pltpu.sync_copy(col2_h.at[wid], ic)
        pltpu.sync_copy(ones_h, bo)
        plsc.subcore_barrier()

        def body(j, carry):
            pltpu.sync_copy(bo, acc.at[ic.at[j]], add=True)
            return carry

        lax.fori_loop(0, _NCHUNK, body, 0)
        plsc.subcore_barrier()
        pltpu.sync_copy(acc.at[pl.ds(sid * _TS, _TS)],
                        out_h.at[pl.ds(cid * _NP + sid * _TS, _TS)])

    return k


def _sc_count(col2, zeros_np, ones_ch):
    return _sc_count_kernel()(col2, zeros_np, ones_ch)


def _edge_body(first, e_ref, grow_ref, gcol_ref, w0_ref, b0_ref,
               we_ref, be_ref, ae_ref, b1_ref, eo_ref, mo_ref, so_ref):
    @pl.when(pl.program_id(0) == 0)
    def _():
        so_ref[...] = jnp.zeros_like(so_ref)

    if first:
        e = jnp.maximum(
            jnp.dot(e_ref[...], w0_ref[...], preferred_element_type=jnp.float32)
            + b0_ref[...], 0.0)
    else:
        st = w0_ref[...]
        mu = st[0:1] / _E
        var = st[1:2] / _E - mu * mu
        scale = st[2:3] * lax.rsqrt(var + 1e-5)
        shift = b0_ref[...] - mu * scale
        e = e_ref[...] * scale + shift

    g = grow_ref[...]
    e2 = jnp.maximum(
        g[:, :_H] + gcol_ref[...]
        + jnp.dot(e, we_ref[...], preferred_element_type=jnp.float32)
        + be_ref[...], 0.0)
    m = jnp.maximum(
        g[:, _H:]
        + jnp.dot(e2, ae_ref[...], preferred_element_type=jnp.float32)
        + b1_ref[...], 0.0)
    eo_ref[...] = e2
    mo_ref[...] = m
    so_ref[...] += jnp.concatenate(
        [jnp.sum(e2, 0, keepdims=True), jnp.sum(e2 * e2, 0, keepdims=True)], 0)


def _edge_call(first, e_in, grow, gcol, w0, b0, we, be, ae, b1):
    d0 = e_in.shape[1]
    full = lambda s: pl.BlockSpec(s, lambda i: (0, 0))
    return pl.pallas_call(
        functools.partial(_edge_body, first),
        grid=(_GE,),
        in_specs=[
            pl.BlockSpec((_BE, d0), lambda i: (i, 0)),
            pl.BlockSpec((_BE, 2 * _H), lambda i: (i, 0)),
            pl.BlockSpec((_BE, _H), lambda i: (i, 0)),
            full(w0.shape),
            full(b0.shape),
            full((_H, _H)),
            full((1, _H)),
            full((_H, _H)),
            full((1, _H)),
        ],
        out_specs=[
            pl.BlockSpec((_BE, _H), lambda i: (i, 0)),
            pl.BlockSpec((_BE, _H), lambda i: (i, 0)),
            pl.BlockSpec((2, _H), lambda i: (0, 0)),
        ],
        out_shape=[
            jax.ShapeDtypeStruct((_E, _H), jnp.float32),
            jax.ShapeDtypeStruct((_E, _H), jnp.float32),
            jax.ShapeDtypeStruct((2, _H), jnp.float32),
        ],
    )(e_in, grow, gcol, w0, b0, we, be, ae, b1)


def _node0_body(x_ref, wn_ref, bn_ref, wrow_ref, wcol_ref,
                xo_ref, prow_ref, pcol_ref):
    x1 = jnp.maximum(
        jnp.dot(x_ref[...], wn_ref[...], preferred_element_type=jnp.float32)
        + bn_ref[...], 0.0)
    xo_ref[...] = x1
    prow_ref[...] = jnp.dot(x1, wrow_ref[...], preferred_element_type=jnp.float32)
    pcol_ref[...] = jnp.dot(x1, wcol_ref[...], preferred_element_type=jnp.float32)


def _node0_call(x, wn, bn, wrow, wcol):
    return pl.pallas_call(
        _node0_body,
        out_shape=[
            jax.ShapeDtypeStruct((_N, _H), jnp.float32),
            jax.ShapeDtypeStruct((_N, 2 * _H), jnp.float32),
            jax.ShapeDtypeStruct((_N, _H), jnp.float32),
        ],
    )(x, wn, bn, wrow, wcol)


def _agg_x(x_ref, sp_ref, cp_ref, bx_ref, bagg_ref, b2_ref):
    cnt = cp_ref[0:_N, 0:1] + cp_ref[_NP:_NP + _N, 0:1]
    s = sp_ref[0:_N, :] + sp_ref[_NP:_NP + _N, :]
    agg = s / jnp.maximum(cnt, 1.0)
    return jnp.maximum(
        jnp.dot(x_ref[...], bx_ref[...], preferred_element_type=jnp.float32)
        + jnp.dot(agg, bagg_ref[...], preferred_element_type=jnp.float32)
        + b2_ref[...], 0.0)


def _node_mid_body(x_ref, sp_ref, cp_ref, bx_ref, bagg_ref, b2_ref,
                   g_ref, b_ref, wrow_ref, wcol_ref,
                   xo_ref, prow_ref, pcol_ref):
    xn = _agg_x(x_ref, sp_ref, cp_ref, bx_ref, bagg_ref, b2_ref)
    mu = jnp.mean(xn, 0, keepdims=True)
    var = jnp.mean(xn * xn, 0, keepdims=True) - mu * mu
    xb = (xn - mu) * (g_ref[...] * lax.rsqrt(var + 1e-5)) + b_ref[...]
    xo_ref[...] = xb
    prow_ref[...] = jnp.dot(xb, wrow_ref[...], preferred_element_type=jnp.float32)
    pcol_ref[...] = jnp.dot(xb, wcol_ref[...], preferred_element_type=jnp.float32)


def _node_mid_call(x, sp, cp, bx, bagg, b2, g, b, wrow, wcol):
    return pl.pallas_call(
        _node_mid_body,
        out_shape=[
            jax.ShapeDtypeStruct((_N, _H), jnp.float32),
            jax.ShapeDtypeStruct((_N, 2 * _H), jnp.float32),
            jax.ShapeDtypeStruct((_N, _H), jnp.float32),
        ],
    )(x, sp, cp, bx, bagg, b2, g, b, wrow, wcol)


def _node_fin_body(x_ref, sp_ref, cp_ref, bx_ref, bagg_ref, b2_ref,
                   st_ref, rw_ref, rb_ref, out_ref):
    xn = _agg_x(x_ref, sp_ref, cp_ref, bx_ref, bagg_ref, b2_ref)
    node_r = jnp.mean(xn, 0, keepdims=True)
    edge_r = st_ref[0:1] / _E
    out_ref[...] = (
        jnp.dot(node_r, rw_ref[0:_H], preferred_element_type=jnp.float32)
        + jnp.dot(edge_r, rw_ref[_H:], preferred_element_type=jnp.float32)
        + rb_ref[...])


def _node_fin_call(x, sp, cp, bx, bagg, b2, st, rw, rb):
    return pl.pallas_call(
        _node_fin_body,
        out_shape=jax.ShapeDtypeStruct((1, 1), jnp.float32),
    )(x, sp, cp, bx, bagg, b2, st, rw, rb)


def kernel(x, edge_index, edge_attr, params):
    row = edge_index[0]
    col = edge_index[1]
    col2 = col.reshape(_NW, _NCHUNK, _CH)
    layers = params['layers']

    zeros_np = jnp.zeros((_NP, _H), jnp.float32)
    ones_ch = jnp.ones((_CH, _H), jnp.float32)
    cp = _sc_count(col2, zeros_np, ones_ch)

    def r1(v):
        return v.reshape(1, -1)

    def tables(layer):
        wrow = jnp.concatenate(
            [layer['edge_W'][:_H], layer['n1_W'][:_H]], axis=1)
        return wrow, layer['edge_W'][_H:2 * _H]

    wrow, wcol = tables(layers[0])
    xc, prow, pcol = _node0_call(
        x, params['enc_node_W'], r1(params['enc_node_b']), wrow, wcol)

    e = edge_attr
    stats = None
    out = None
    for i, li in enumerate(layers):
        grow, gcol = _sc_gather(row, col, prow, pcol)
        we = li['edge_W'][2 * _H:]
        ae = li['n1_W'][_H:]
        if i == 0:
            w0 = params['enc_edge_W']
            b0 = r1(params['enc_edge_b'])
        else:
            w0 = jnp.concatenate([stats, r1(params['edge_norm_g'])], axis=0)
            b0 = r1(params['edge_norm_b'])
        e, m, stats = _edge_call(i == 0, e, grow, gcol, w0, b0,
                                 we, r1(li['edge_b']), ae, r1(li['n1_b']))
        sp = _sc_scatter(m, col2, zeros_np)
        bx = li['n2_W'][:_H]
        bagg = li['n2_W'][_H:]
        if i < len(layers) - 1:
            wrow, wcol = tables(layers[i + 1])
            xc, prow, pcol = _node_mid_call(
                xc, sp, cp, bx, bagg, r1(li['n2_b']),
                r1(params['node_norm_g']), r1(params['node_norm_b']),
                wrow, wcol)
        else:
            out = _node_fin_call(
                xc, sp, cp, bx, bagg, r1(li['n2_b']),
                stats, params['reg_W'], r1(params['reg_b']))
    return out
